# baseline (speedup 1.0000x reference)
import functools

import jax
import jax.numpy as jnp
from jax import lax
from jax.experimental import pallas as pl
from jax.experimental.pallas import tpu as pltpu
from jax.experimental.pallas import tpu_sc as plsc

N = 10000
E = 320000
C = 128
NC = 2
NS = 16
NW = NC * NS
EW = E // NW
CH = 80
NCHUNK = EW // CH
NSLOT = 4
EIX = 6
GLA = 2
PERIOD = 12
RS = 624
RTAIL = N - NS * RS


def _prep_body(x_ref, w_ref, alv_ref, arv_ref, h_ref, al_ref, ar_ref, ps_ref):
    x = x_ref[...]
    w = w_ref[...]
    h = lax.dot_general(x, w, (((1,), (1,)), ((), ())),
                        preferred_element_type=jnp.float32)
    h_ref[...] = h
    al = lax.dot_general(h, alv_ref[...], (((1,), (0,)), ((), ())),
                         preferred_element_type=jnp.float32)
    ar = lax.dot_general(h, arv_ref[...], (((1,), (0,)), ((), ())),
                         preferred_element_type=jnp.float32)
    al_ref[...] = al
    ar_ref[...] = ar
    a = al + ar
    a = jnp.where(a >= 0.0, a, 0.2 * a)
    ps_ref[...] = jnp.exp(a)


_prep = pl.pallas_call(
    _prep_body,
    out_shape=[
        jax.ShapeDtypeStruct((N, C), jnp.float32),
        jax.ShapeDtypeStruct((N, 1), jnp.float32),
        jax.ShapeDtypeStruct((N, 1), jnp.float32),
        jax.ShapeDtypeStruct((N, 1), jnp.float32),
    ],
)


def _sc_body(h_hbm, ei_hbm, al_hbm, ar_hbm, z1_hbm,
             acc_out, s_out,
             rows, pv, alg, arg, eidx, gsem, ssem, psem, isem,
             acc_sh, s_sh):
    cid = lax.axis_index("c")
    sid = lax.axis_index("s")
    wid = sid * NC + cid
    cbase = wid * NCHUNK

    z16 = jnp.zeros((16,), jnp.float32)

    @plsc.parallel_loop(0, CH, unroll=4)
    def _(r):
        for g in range(C // 16):
            rows[0][r, pl.ds(g * 16, 16)] = z16

    rb = sid * RS
    for k in range(RS // CH):
        pltpu.async_copy(rows[0], acc_sh.at[pl.ds(rb + k * CH, CH)], gsem[0])
    pltpu.async_copy(rows[0].at[pl.ds(0, RS % CH)],
                     acc_sh.at[pl.ds(rb + (RS // CH) * CH, RS % CH)], gsem[0])

    @pl.when(sid == NS - 1)
    def _():
        pltpu.async_copy(rows[0].at[pl.ds(0, RTAIL)],
                         acc_sh.at[pl.ds(NS * RS, RTAIL)], gsem[0])

    @pl.when(sid == 0)
    def _():
        pltpu.sync_copy(z1_hbm, s_sh)

    for k in range(RS // CH):
        pltpu.make_async_copy(rows[0], acc_sh.at[pl.ds(rb + k * CH, CH)],
                              gsem[0]).wait()
    pltpu.make_async_copy(rows[0].at[pl.ds(0, RS % CH)],
                          acc_sh.at[pl.ds(rb + (RS // CH) * CH, RS % CH)],
                          gsem[0]).wait()

    @pl.when(sid == NS - 1)
    def _():
        pltpu.make_async_copy(rows[0].at[pl.ds(0, RTAIL)],
                              acc_sh.at[pl.ds(NS * RS, RTAIL)], gsem[0]).wait()

    plsc.subcore_barrier()

    def idx_fire(c, e):
        pltpu.async_copy(ei_hbm.at[:, cbase + c, :], eidx[e], isem[e])

    def gather_fire(c, s, e, first=False):
        if not first:
            ep = (e + EIX - NSLOT) % EIX

            @pl.when(c >= NSLOT)
            def _():
                pltpu.make_async_copy(rows[s], acc_sh.at[eidx[ep].at[1]],
                                      ssem[s]).wait()
                pltpu.make_async_copy(pv[s], s_sh.at[eidx[ep].at[1]],
                                      psem[s]).wait()
        pltpu.make_async_copy(ei_hbm.at[:, cbase + c, :], eidx[e],
                              isem[e]).wait()
        pltpu.async_copy(h_hbm.at[eidx[e].at[0]], rows[s], gsem[s])
        pltpu.async_copy(al_hbm.at[eidx[e].at[0]], alg[s], gsem[s])
        pltpu.async_copy(ar_hbm.at[eidx[e].at[1]], arg[s], gsem[s])

    def process(c, s, e):
        pltpu.make_async_copy(h_hbm.at[eidx[e].at[0]], rows[s], gsem[s]).wait()
        pltpu.make_async_copy(al_hbm.at[eidx[e].at[0]], alg[s], gsem[s]).wait()
        pltpu.make_async_copy(ar_hbm.at[eidx[e].at[1]], arg[s], gsem[s]).wait()

        for g in range(CH // 16):
            a = alg[s][pl.ds(g * 16, 16)] + arg[s][pl.ds(g * 16, 16)]
            a = jnp.where(a >= 0.0, a, 0.2 * a)
            pv[s][pl.ds(g * 16, 16)] = jnp.exp(a)
        pltpu.async_copy(pv[s], s_sh.at[eidx[e].at[1]], psem[s], add=True)

        @plsc.parallel_loop(0, CH, unroll=4)
        def _(r):
            pb = plsc.load_gather(pv[s], [jnp.full((16,), r, jnp.int32)])
            for g in range(C // 16):
                rows[s][r, pl.ds(g * 16, 16)] = (
                    rows[s][r, pl.ds(g * 16, 16)] * pb)

        pltpu.async_copy(rows[s], acc_sh.at[eidx[e].at[1]], ssem[s], add=True)

        @pl.when(c + GLA + 1 < NCHUNK)
        def _():
            idx_fire(c + GLA + 1, (e + GLA + 1) % EIX)

        @pl.when(c + GLA < NCHUNK)
        def _():
            gather_fire(c + GLA, (s + GLA) % NSLOT, (e + GLA) % EIX)

    for c0 in range(GLA + 1):
        idx_fire(c0, c0)
    for c0 in range(GLA):
        gather_fire(c0, c0, c0, first=True)

    def period_body(i, carry):
        for k in range(PERIOD):
            process(PERIOD * i + k, k % NSLOT, k % EIX)
        return carry

    _NB = (NCHUNK - GLA - 1) // PERIOD
    lax.fori_loop(0, _NB, period_body, 0)
    for c in range(_NB * PERIOD, NCHUNK):
        process(c, c % NSLOT, c % EIX)

    for c in range(NCHUNK - NSLOT, NCHUNK):
        s, e = c % NSLOT, c % EIX
        pltpu.make_async_copy(rows[s], acc_sh.at[eidx[e].at[1]],
                              ssem[s]).wait()
        pltpu.make_async_copy(pv[s], s_sh.at[eidx[e].at[1]], psem[s]).wait()

    plsc.subcore_barrier()

    pltpu.sync_copy(acc_sh.at[pl.ds(rb, RS)], acc_out.at[cid, pl.ds(rb, RS)])

    @pl.when(sid == NS - 1)
    def _():
        pltpu.sync_copy(acc_sh.at[pl.ds(NS * RS, RTAIL)],
                        acc_out.at[cid, pl.ds(NS * RS, RTAIL)])

    @pl.when(sid == 0)
    def _():
        pltpu.sync_copy(s_sh, s_out.at[cid])


def _sc_entry(h_hbm, ei_hbm, al_hbm, ar_hbm, z1_hbm,
              acc_out, s_out,
              rows0, rows1, rows2, rows3, p0, p1, p2, p3,
              ag0, ag1, ag2, ag3, ar0, ar1, ar2, ar3,
              ei0, ei1, ei2, ei3, ei4, ei5,
              gs0, gs1, gs2, gs3, ss0, ss1, ss2, ss3,
              ps0, ps1, ps2, ps3,
              is0, is1, is2, is3, is4, is5,
              acc_sh, s_sh):
    _sc_body(h_hbm, ei_hbm, al_hbm, ar_hbm, z1_hbm,
             acc_out, s_out,
             (rows0, rows1, rows2, rows3), (p0, p1, p2, p3),
             (ag0, ag1, ag2, ag3), (ar0, ar1, ar2, ar3),
             (ei0, ei1, ei2, ei3, ei4, ei5),
             (gs0, gs1, gs2, gs3), (ss0, ss1, ss2, ss3),
             (ps0, ps1, ps2, ps3),
             (is0, is1, is2, is3, is4, is5),
             acc_sh, s_sh)
_sc = pl.kernel(
    _sc_entry,
    mesh=plsc.VectorSubcoreMesh(core_axis_name="c", subcore_axis_name="s"),
    compiler_params=pltpu.CompilerParams(needs_layout_passes=False,
                                         disable_bounds_checks=True,
                                         disable_semaphore_checks=True),
    out_type=[
        jax.ShapeDtypeStruct((NC, N, C), jnp.float32),
        jax.ShapeDtypeStruct((NC, N), jnp.float32),
    ],
    scratch_types=(
        [pltpu.VMEM((CH, C), jnp.float32)] * NSLOT
        + [pltpu.VMEM((CH,), jnp.float32)] * NSLOT
        + [pltpu.VMEM((CH,), jnp.float32)] * (2 * NSLOT)
        + [pltpu.VMEM((2, CH), jnp.int32)] * EIX
        + [pltpu.SemaphoreType.DMA] * (3 * NSLOT)
        + [pltpu.SemaphoreType.DMA] * EIX
        + [pltpu.VMEM_SHARED((N, C), jnp.float32),
           pltpu.VMEM_SHARED((N,), jnp.float32)]
    ),
)


def _fin_body(acc_ref, s_ref, ps_ref, h_ref, o_ref):
    ps = ps_ref[...]
    num = acc_ref[0] + acc_ref[1] + ps * h_ref[...]
    den = s_ref[0] + s_ref[1] + ps
    o_ref[...] = num / jnp.maximum(den, 1e-6)


_BR = 2000

_fin = pl.pallas_call(
    _fin_body,
    grid=(N // _BR,),
    in_specs=[
        pl.BlockSpec((NC, _BR, C), lambda i: (0, i, 0)),
        pl.BlockSpec((NC, _BR, 1), lambda i: (0, i, 0)),
        pl.BlockSpec((_BR, 1), lambda i: (i, 0)),
        pl.BlockSpec((_BR, C), lambda i: (i, 0)),
    ],
    out_specs=pl.BlockSpec((_BR, C), lambda i: (i, 0)),
    out_shape=jax.ShapeDtypeStruct((N, C), jnp.float32),
)


def kernel(x, edge_index, W, attn_l, attn_r):
    alv = attn_l.reshape(C, 1).astype(jnp.float32)
    arv = attn_r.reshape(C, 1).astype(jnp.float32)
    h, al2, ar2, ps = _prep(x, W, alv, arv)

    ei3 = edge_index.reshape(2, E // CH, CH)
    al = al2.reshape(N)
    ar = ar2.reshape(N)
    z1 = jnp.zeros((N,), jnp.float32)

    acc2, s2 = _sc(h, ei3, al, ar, z1)

    return _fin(acc2, s2.reshape(NC, N, 1), ps, h)


# speedup vs baseline: 1.0848x; 1.0848x over previous
import functools

import jax
import jax.numpy as jnp
from jax import lax
from jax.experimental import pallas as pl
from jax.experimental.pallas import tpu as pltpu
from jax.experimental.pallas import tpu_sc as plsc

N = 10000
E = 320000
C = 128
NC = 2
NS = 16
NW = NC * NS
EW = E // NW
CH = 80
NCHUNK = EW // CH
NSLOT = 4
EIX = 6
GLA = 3
PERIOD = 12
RS = 624
RTAIL = N - NS * RS


def _prep_body(x_ref, w_ref, alv_ref, arv_ref, h_ref, al_ref, ar_ref, ps_ref):
    x = x_ref[...]
    w = w_ref[...]
    h = lax.dot_general(x, w, (((1,), (1,)), ((), ())),
                        preferred_element_type=jnp.float32)
    h_ref[...] = h
    al = lax.dot_general(h, alv_ref[...], (((1,), (0,)), ((), ())),
                         preferred_element_type=jnp.float32)
    ar = lax.dot_general(h, arv_ref[...], (((1,), (0,)), ((), ())),
                         preferred_element_type=jnp.float32)
    al_ref[...] = al
    ar_ref[...] = ar
    a = al + ar
    a = jnp.where(a >= 0.0, a, 0.2 * a)
    ps_ref[...] = jnp.exp(a)


_prep = pl.pallas_call(
    _prep_body,
    out_shape=[
        jax.ShapeDtypeStruct((N, C), jnp.float32),
        jax.ShapeDtypeStruct((N, 1), jnp.float32),
        jax.ShapeDtypeStruct((N, 1), jnp.float32),
        jax.ShapeDtypeStruct((N, 1), jnp.float32),
    ],
)


def _sc_body(h_hbm, ei_hbm, al_hbm, ar_hbm, z1_hbm,
             acc_out, s_out,
             rows, pv, alg, arg, eidx, gsem, ssem, psem, isem,
             acc_sh, s_sh):
    cid = lax.axis_index("c")
    sid = lax.axis_index("s")
    wid = sid * NC + cid
    cbase = wid * NCHUNK

    z16 = jnp.zeros((16,), jnp.float32)

    @plsc.parallel_loop(0, CH, unroll=4)
    def _(r):
        for g in range(C // 16):
            rows[0][r, pl.ds(g * 16, 16)] = z16

    rb = sid * RS
    for k in range(RS // CH):
        pltpu.async_copy(rows[0], acc_sh.at[pl.ds(rb + k * CH, CH)], gsem[0])
    pltpu.async_copy(rows[0].at[pl.ds(0, RS % CH)],
                     acc_sh.at[pl.ds(rb + (RS // CH) * CH, RS % CH)], gsem[0])

    @pl.when(sid == NS - 1)
    def _():
        pltpu.async_copy(rows[0].at[pl.ds(0, RTAIL)],
                         acc_sh.at[pl.ds(NS * RS, RTAIL)], gsem[0])

    @pl.when(sid == 0)
    def _():
        pltpu.sync_copy(z1_hbm, s_sh)

    for k in range(RS // CH):
        pltpu.make_async_copy(rows[0], acc_sh.at[pl.ds(rb + k * CH, CH)],
                              gsem[0]).wait()
    pltpu.make_async_copy(rows[0].at[pl.ds(0, RS % CH)],
                          acc_sh.at[pl.ds(rb + (RS // CH) * CH, RS % CH)],
                          gsem[0]).wait()

    @pl.when(sid == NS - 1)
    def _():
        pltpu.make_async_copy(rows[0].at[pl.ds(0, RTAIL)],
                              acc_sh.at[pl.ds(NS * RS, RTAIL)], gsem[0]).wait()

    plsc.subcore_barrier()

    def idx_fire(c, e):
        pltpu.async_copy(ei_hbm.at[:, cbase + c, :], eidx[e], isem[e])

    def gather_fire(c, s, e, first=False):
        if not first:
            ep = (e + EIX - NSLOT) % EIX

            @pl.when(c >= NSLOT)
            def _():
                pltpu.make_async_copy(rows[s], acc_sh.at[eidx[ep].at[1]],
                                      ssem[s]).wait()
                pltpu.make_async_copy(pv[s], s_sh.at[eidx[ep].at[1]],
                                      psem[s]).wait()
        pltpu.make_async_copy(ei_hbm.at[:, cbase + c, :], eidx[e],
                              isem[e]).wait()
        pltpu.async_copy(h_hbm.at[eidx[e].at[0]], rows[s], gsem[s])
        pltpu.async_copy(al_hbm.at[eidx[e].at[0]], alg[s], gsem[s])
        pltpu.async_copy(ar_hbm.at[eidx[e].at[1]], arg[s], gsem[s])

    def process(c, s, e):
        pltpu.make_async_copy(h_hbm.at[eidx[e].at[0]], rows[s], gsem[s]).wait()
        pltpu.make_async_copy(al_hbm.at[eidx[e].at[0]], alg[s], gsem[s]).wait()
        pltpu.make_async_copy(ar_hbm.at[eidx[e].at[1]], arg[s], gsem[s]).wait()

        for g in range(CH // 16):
            a = alg[s][pl.ds(g * 16, 16)] + arg[s][pl.ds(g * 16, 16)]
            a = jnp.where(a >= 0.0, a, 0.2 * a)
            pv[s][pl.ds(g * 16, 16)] = jnp.exp(a)
        pltpu.async_copy(pv[s], s_sh.at[eidx[e].at[1]], psem[s], add=True)

        @plsc.parallel_loop(0, CH, unroll=4)
        def _(r):
            pb = plsc.load_gather(pv[s], [jnp.full((16,), r, jnp.int32)])
            for g in range(C // 16):
                rows[s][r, pl.ds(g * 16, 16)] = (
                    rows[s][r, pl.ds(g * 16, 16)] * pb)

        pltpu.async_copy(rows[s], acc_sh.at[eidx[e].at[1]], ssem[s], add=True)

        @pl.when(c + GLA + 1 < NCHUNK)
        def _():
            idx_fire(c + GLA + 1, (e + GLA + 1) % EIX)

        @pl.when(c + GLA < NCHUNK)
        def _():
            gather_fire(c + GLA, (s + GLA) % NSLOT, (e + GLA) % EIX)

    for c0 in range(GLA + 1):
        idx_fire(c0, c0)
    for c0 in range(GLA):
        gather_fire(c0, c0, c0, first=True)

    def period_body(i, carry):
        for k in range(PERIOD):
            process(PERIOD * i + k, k % NSLOT, k % EIX)
        return carry

    _NB = (NCHUNK - GLA - 1) // PERIOD
    lax.fori_loop(0, _NB, period_body, 0)
    for c in range(_NB * PERIOD, NCHUNK):
        process(c, c % NSLOT, c % EIX)

    for c in range(NCHUNK - NSLOT, NCHUNK):
        s, e = c % NSLOT, c % EIX
        pltpu.make_async_copy(rows[s], acc_sh.at[eidx[e].at[1]],
                              ssem[s]).wait()
        pltpu.make_async_copy(pv[s], s_sh.at[eidx[e].at[1]], psem[s]).wait()

    plsc.subcore_barrier()

    pltpu.sync_copy(acc_sh.at[pl.ds(rb, RS)], acc_out.at[cid, pl.ds(rb, RS)])

    @pl.when(sid == NS - 1)
    def _():
        pltpu.sync_copy(acc_sh.at[pl.ds(NS * RS, RTAIL)],
                        acc_out.at[cid, pl.ds(NS * RS, RTAIL)])

    @pl.when(sid == 0)
    def _():
        pltpu.sync_copy(s_sh, s_out.at[cid])


def _sc_entry(h_hbm, ei_hbm, al_hbm, ar_hbm, z1_hbm,
              acc_out, s_out,
              rows0, rows1, rows2, rows3, p0, p1, p2, p3,
              ag0, ag1, ag2, ag3, ar0, ar1, ar2, ar3,
              ei0, ei1, ei2, ei3, ei4, ei5,
              gs0, gs1, gs2, gs3, ss0, ss1, ss2, ss3,
              ps0, ps1, ps2, ps3,
              is0, is1, is2, is3, is4, is5,
              acc_sh, s_sh):
    _sc_body(h_hbm, ei_hbm, al_hbm, ar_hbm, z1_hbm,
             acc_out, s_out,
             (rows0, rows1, rows2, rows3), (p0, p1, p2, p3),
             (ag0, ag1, ag2, ag3), (ar0, ar1, ar2, ar3),
             (ei0, ei1, ei2, ei3, ei4, ei5),
             (gs0, gs1, gs2, gs3), (ss0, ss1, ss2, ss3),
             (ps0, ps1, ps2, ps3),
             (is0, is1, is2, is3, is4, is5),
             acc_sh, s_sh)
_sc = pl.kernel(
    _sc_entry,
    mesh=plsc.VectorSubcoreMesh(core_axis_name="c", subcore_axis_name="s"),
    compiler_params=pltpu.CompilerParams(needs_layout_passes=False,
                                         disable_bounds_checks=True,
                                         disable_semaphore_checks=True),
    out_type=[
        jax.ShapeDtypeStruct((NC, N, C), jnp.float32),
        jax.ShapeDtypeStruct((NC, N), jnp.float32),
    ],
    scratch_types=(
        [pltpu.VMEM((CH, C), jnp.float32)] * NSLOT
        + [pltpu.VMEM((CH,), jnp.float32)] * NSLOT
        + [pltpu.VMEM((CH,), jnp.float32)] * (2 * NSLOT)
        + [pltpu.VMEM((2, CH), jnp.int32)] * EIX
        + [pltpu.SemaphoreType.DMA] * (3 * NSLOT)
        + [pltpu.SemaphoreType.DMA] * EIX
        + [pltpu.VMEM_SHARED((N, C), jnp.float32),
           pltpu.VMEM_SHARED((N,), jnp.float32)]
    ),
)


def _fin_body(acc_ref, s_ref, ps_ref, h_ref, o_ref):
    ps = ps_ref[...]
    num = acc_ref[0] + acc_ref[1] + ps * h_ref[...]
    den = s_ref[0] + s_ref[1] + ps
    o_ref[...] = num / jnp.maximum(den, 1e-6)


_BR = 2000

_fin = pl.pallas_call(
    _fin_body,
    grid=(N // _BR,),
    in_specs=[
        pl.BlockSpec((NC, _BR, C), lambda i: (0, i, 0)),
        pl.BlockSpec((NC, _BR, 1), lambda i: (0, i, 0)),
        pl.BlockSpec((_BR, 1), lambda i: (i, 0)),
        pl.BlockSpec((_BR, C), lambda i: (i, 0)),
    ],
    out_specs=pl.BlockSpec((_BR, C), lambda i: (i, 0)),
    out_shape=jax.ShapeDtypeStruct((N, C), jnp.float32),
)


def kernel(x, edge_index, W, attn_l, attn_r):
    alv = attn_l.reshape(C, 1).astype(jnp.float32)
    arv = attn_r.reshape(C, 1).astype(jnp.float32)
    h, al2, ar2, ps = _prep(x, W, alv, arv)

    ei3 = edge_index.reshape(2, E // CH, CH)
    al = al2.reshape(N)
    ar = ar2.reshape(N)
    z1 = jnp.zeros((N,), jnp.float32)

    acc2, s2 = _sc(h, ei3, al, ar, z1)

    return _fin(acc2, s2.reshape(NC, N, 1), ps, h)
